# baseline (speedup 1.0000x reference)
import functools

import jax
import jax.numpy as jnp
from jax import lax
from jax.experimental import pallas as pl
from jax.experimental.pallas import tpu as pltpu
from jax.experimental.pallas import tpu_sc as plsc

_SC_STUB = False
MAXK = 4096
LOGN = 15
NPAD = 1 << LOGN
LANES = 128
SUB = NPAD // LANES

NW = 32
CHUNK = 128
ROWS_PER_W = (8 * MAXK) // NW
CPW = ROWS_PER_W // CHUNK


def _stage_consts():
    js, ks = [], []
    for p in range(1, LOGN + 1):
        k = 1 << p
        for q in range(p - 1, -1, -1):
            js.append(1 << q)
            ks.append(k)
    return jnp.array(js, jnp.int32), jnp.array(ks, jnp.int32)


def _partner_dyn(x, j, jl, jr, is_left, is_lane):
    up_l = pltpu.roll(x, (LANES - jl) & (LANES - 1), 2)
    dn_l = pltpu.roll(x, jl, 2)
    up_s = pltpu.roll(x, (SUB - jr) & (SUB - 1), 1)
    dn_s = pltpu.roll(x, jr, 1)
    up = jnp.where(is_lane, up_l, up_s)
    dn = jnp.where(is_lane, dn_l, dn_s)
    return jnp.where(is_left, up, dn)


def _sort_body(j_ref, k_ref, key_ref, out_key_ref, out_idx_ref):
    shape = (key_ref.shape[0], SUB, LANES)
    s_io = lax.broadcasted_iota(jnp.int32, shape, 1)
    l_io = lax.broadcasted_iota(jnp.int32, shape, 2)
    i_io = s_io * LANES + l_io
    out_key_ref[...] = key_ref[...]
    out_idx_ref[...] = i_io

    def stage(s, _):
        j = j_ref[s]
        k = k_ref[s]
        jl = j & (LANES - 1)
        jr = j >> 7
        is_lane = jnp.broadcast_to(j < LANES, shape)
        is_left = (i_io & j) == 0
        d = (i_io & k) == 0
        key = out_key_ref[...]
        idx = out_idx_ref[...]
        pk = _partner_dyn(key, j, jl, jr, is_left, is_lane)
        pi = _partner_dyn(idx, j, jl, jr, is_left, is_lane)
        prec = (key > pk) | ((key == pk) & (idx < pi))
        keep = prec == (is_left == d)
        out_key_ref[...] = jnp.where(keep, key, pk)
        out_idx_ref[...] = jnp.where(keep, idx, pi)
        return _

    lax.fori_loop(0, j_ref.shape[0], stage, None)


def _topk_sort(scores):
    B, N = scores.shape
    pad = jnp.full((B, NPAD - N), -1.0, dtype=scores.dtype)
    keys = jnp.concatenate([scores, pad], axis=1).reshape(B, SUB, LANES)
    js, ks = _stage_consts()
    skey, sidx = pl.pallas_call(
        _sort_body,
        in_specs=[
            pl.BlockSpec(memory_space=pltpu.SMEM),
            pl.BlockSpec(memory_space=pltpu.SMEM),
            pl.BlockSpec(),
        ],
        out_specs=(
            pl.BlockSpec(),
            pl.BlockSpec(),
        ),
        out_shape=(
            jax.ShapeDtypeStruct((B, SUB, LANES), jnp.float32),
            jax.ShapeDtypeStruct((B, SUB, LANES), jnp.int32),
        ),
    )(js, ks, keys)
    return skey.reshape(B, NPAD), sidx.reshape(B, NPAD)


def _make_sc_gather(n_kp, d_desc, d_kp):
    mesh = plsc.VectorSubcoreMesh(core_axis_name="c", subcore_axis_name="s")

    @functools.partial(
        pl.kernel,
        mesh=mesh,
        compiler_params=pltpu.CompilerParams(needs_layout_passes=False),
        out_type=(
            jax.ShapeDtypeStruct((NW * ROWS_PER_W, d_desc), jnp.float32),
            jax.ShapeDtypeStruct((NW * ROWS_PER_W * d_kp,), jnp.float32),
        ),
        scratch_types=[
            pltpu.VMEM((CPW, CHUNK), jnp.int32),
            pltpu.VMEM((ROWS_PER_W,), jnp.int32),
            pltpu.VMEM((CHUNK, d_desc), jnp.float32),
            pltpu.VMEM((n_kp * d_kp,), jnp.float32),
            pltpu.VMEM((ROWS_PER_W * d_kp,), jnp.float32),
            pltpu.SemaphoreType.DMA,
        ],
    )
    def gather_k(desc_hbm, kp_hbm, gidx_hbm, lidx_hbm, desc_out, kp_out,
                 gidx_v, lidx_v, rows_v, kp_tab_v, kp_res_v, sem_d):
        wid = lax.axis_index("s") * 2 + lax.axis_index("c")
        r = wid // (NW // 8)
        base = wid * ROWS_PER_W
        pltpu.sync_copy(gidx_hbm.at[wid], gidx_v)
        pltpu.sync_copy(lidx_hbm.at[wid], lidx_v)
        pltpu.sync_copy(kp_hbm.at[r], kp_tab_v)
        io16 = lax.iota(jnp.int32, 16)
        for ch in range(CPW):
            cd = pltpu.async_copy(desc_hbm.at[gidx_v.at[ch]], rows_v, sem_d)
            for t in range(CHUNK // 16):
                o = ch * CHUNK + t * 16
                lvec = lidx_v[pl.ds(o, 16)] * 2
                xs = plsc.load_gather(kp_tab_v, [lvec])
                ys = plsc.load_gather(kp_tab_v, [lvec + 1])
                ovec = (io16 + o) * 2
                plsc.store_scatter(kp_res_v, [ovec], xs)
                plsc.store_scatter(kp_res_v, [ovec + 1], ys)
            cd.wait()
            pltpu.sync_copy(rows_v, desc_out.at[pl.ds(base + ch * CHUNK, CHUNK)])
        pltpu.sync_copy(kp_res_v, kp_out.at[pl.ds(base * 2, ROWS_PER_W * 2)])

    return gather_k


def kernel(keypoints, scores, descriptors):
    B, N, D = descriptors.shape
    skey, sidx = _topk_sort(scores)
    scores_top = skey[:, :MAXK]
    idx_top = sidx[:, :MAXK]

    gidx = (idx_top + (jnp.arange(B, dtype=jnp.int32) * N)[:, None]).astype(jnp.int32)
    gidx = gidx.reshape(NW, CPW, CHUNK)
    lidx = idx_top.reshape(NW, ROWS_PER_W)

    desc_tab = descriptors.reshape(B * N, D)

    if _SC_STUB:
        desc_flat = jnp.zeros((NW * ROWS_PER_W, D), jnp.float32) + gidx.sum()
        kp_flat = jnp.zeros((NW * ROWS_PER_W * 2,), jnp.float32) + lidx.sum()
    else:
        gather_k = _make_sc_gather(N, D, 2)
        desc_flat, kp_flat = gather_k(desc_tab, keypoints.reshape(B, N * 2), gidx, lidx)

    final_kpts = kp_flat.reshape(B, MAXK, 2)
    final_desc = desc_flat.reshape(B, MAXK, D)
    num_valid = jnp.full((B,), MAXK, dtype=jnp.int32)
    return (final_kpts, scores_top, final_desc, num_valid)


# speedup vs baseline: 3.2585x; 3.2585x over previous
import functools

import jax
import jax.numpy as jnp
from jax import lax
from jax.experimental import pallas as pl
from jax.experimental.pallas import tpu as pltpu
from jax.experimental.pallas import tpu_sc as plsc

MAXK = 4096
LANES = 128
LOGK = 12
KSUB = MAXK // LANES

NW = 32
CHUNK = 128
ROWS_PER_W = (8 * MAXK) // NW
CPW = ROWS_PER_W // CHUNK

ONE_F32_BITS = 0x3F800000


def _thresh_body(key_ref, t_ref, cnt_ref, neq_ref):
    kb = lax.bitcast_convert_type(key_ref[...], jnp.int32)
    B = kb.shape[0]
    lo = jnp.full((B, LANES), -1, jnp.int32)
    hi = jnp.full((B, LANES), ONE_F32_BITS, jnp.int32)

    def count_gt(t):
        c = jnp.sum((kb > t[:, None, :]).astype(jnp.int32), axis=1)
        return jnp.sum(c, axis=1, keepdims=True) + jnp.zeros((B, LANES), jnp.int32)

    def step(_, lohi):
        lo, hi = lohi
        mid = (lo + hi) >> 1
        c = count_gt(mid)
        big = c >= MAXK
        return jnp.where(big, mid, lo), jnp.where(big, hi, mid)

    lo, hi = lax.fori_loop(0, 31, step, (lo, hi))
    cnt = count_gt(hi)
    t_ref[...] = hi
    cnt_ref[...] = cnt
    neq_ref[...] = MAXK - cnt


def _threshold(scores):
    B, N = scores.shape
    S = -(-N // LANES)
    padded = jnp.concatenate(
        [scores, jnp.full((B, S * LANES - N), -1.0, scores.dtype)], axis=1
    ).reshape(B, S, LANES)
    return pl.pallas_call(
        _thresh_body,
        out_shape=(
            jax.ShapeDtypeStruct((B, LANES), jnp.int32),
            jax.ShapeDtypeStruct((B, LANES), jnp.int32),
            jax.ShapeDtypeStruct((B, LANES), jnp.int32),
        ),
    )(padded)


def _make_sc_compact(B, N):
    mesh = plsc.VectorSubcoreMesh(core_axis_name="c", subcore_axis_name="s")
    nv = N // 16

    @functools.partial(
        pl.kernel,
        mesh=mesh,
        compiler_params=pltpu.CompilerParams(needs_layout_passes=False),
        out_type=(
            jax.ShapeDtypeStruct((B, MAXK), jnp.float32),
            jax.ShapeDtypeStruct((B, MAXK), jnp.int32),
        ),
        scratch_types=[
            pltpu.VMEM((N,), jnp.float32),
            pltpu.VMEM((LANES,), jnp.int32),
            pltpu.VMEM((LANES,), jnp.int32),
            pltpu.VMEM((LANES,), jnp.int32),
            pltpu.VMEM((MAXK,), jnp.float32),
            pltpu.VMEM((MAXK,), jnp.int32),
        ],
    )
    def compact_k(s_hbm, t_hbm, cnt_hbm, neq_hbm, sel_s_hbm, sel_i_hbm,
                  row_v, t_v, cnt_v, neq_v, outs_v, outi_v):
        wid = lax.axis_index("s") * 2 + lax.axis_index("c")

        @pl.when(wid < B)
        def _():
            pltpu.sync_copy(s_hbm.at[wid], row_v)
            pltpu.sync_copy(t_hbm.at[wid], t_v)
            pltpu.sync_copy(cnt_hbm.at[wid], cnt_v)
            pltpu.sync_copy(neq_hbm.at[wid], neq_v)
            z16 = jnp.zeros((16,), jnp.int32)
            tsp = t_v[pl.ds(0, 16)]
            cntsp = cnt_v[pl.ds(0, 16)]
            neqsp = neq_v[pl.ds(0, 16)]
            io16 = lax.iota(jnp.int32, 16)

            def body(i, carry):
                offg, offe = carry
                sl = row_v[pl.ds(pl.multiple_of(i * 16, 16), 16)]
                kb = plsc.bitcast(sl, jnp.int32)
                ivec = io16 + i * 16
                m = kb > tsp
                inc = plsc.cumsum(jnp.where(m, 1, 0))
                pos = offg + inc - 1
                plsc.store_scatter(outs_v, [pos], sl, mask=m)
                plsc.store_scatter(outi_v, [pos], ivec, mask=m)
                m2 = kb == tsp
                inc2 = plsc.cumsum(jnp.where(m2, 1, 0))
                rank = offe + inc2 - 1
                m2w = m2 & (rank < neqsp)
                pos2 = cntsp + rank
                plsc.store_scatter(outs_v, [pos2], sl, mask=m2w)
                plsc.store_scatter(outi_v, [pos2], ivec, mask=m2w)
                return (offg + plsc.all_reduce_population_count(m),
                        offe + plsc.all_reduce_population_count(m2))

            lax.fori_loop(0, nv, body, (z16, z16))
            pltpu.sync_copy(outs_v, sel_s_hbm.at[wid])
            pltpu.sync_copy(outi_v, sel_i_hbm.at[wid])

    return compact_k


def _stage_consts():
    js, ks = [], []
    for p in range(1, LOGK + 1):
        k = 1 << p
        for q in range(p - 1, -1, -1):
            js.append(1 << q)
            ks.append(k)
    return jnp.array(js, jnp.int32), jnp.array(ks, jnp.int32)


def _partner_dyn(x, jl, jr, is_left, is_lane):
    up_l = pltpu.roll(x, (LANES - jl) & (LANES - 1), 2)
    dn_l = pltpu.roll(x, jl, 2)
    up_s = pltpu.roll(x, (KSUB - jr) & (KSUB - 1), 1)
    dn_s = pltpu.roll(x, jr, 1)
    up = jnp.where(is_lane, up_l, up_s)
    dn = jnp.where(is_lane, dn_l, dn_s)
    return jnp.where(is_left, up, dn)


def _sort_body(j_ref, k_ref, key_ref, idx_ref, out_key_ref, out_idx_ref):
    shape = key_ref.shape
    s_io = lax.broadcasted_iota(jnp.int32, shape, 1)
    l_io = lax.broadcasted_iota(jnp.int32, shape, 2)
    i_io = s_io * LANES + l_io
    out_key_ref[...] = key_ref[...]
    out_idx_ref[...] = idx_ref[...]

    def stage(s, _):
        j = j_ref[s]
        k = k_ref[s]
        jl = j & (LANES - 1)
        jr = j >> 7
        is_lane = jnp.broadcast_to(j < LANES, shape)
        is_left = (i_io & j) == 0
        d = (i_io & k) == 0
        key = out_key_ref[...]
        idx = out_idx_ref[...]
        pk = _partner_dyn(key, jl, jr, is_left, is_lane)
        pi = _partner_dyn(idx, jl, jr, is_left, is_lane)
        prec = (key > pk) | ((key == pk) & (idx < pi))
        keep = prec == (is_left == d)
        out_key_ref[...] = jnp.where(keep, key, pk)
        out_idx_ref[...] = jnp.where(keep, idx, pi)
        return _

    lax.fori_loop(0, j_ref.shape[0], stage, None)


def _topk_sort(sel_s, sel_i):
    B = sel_s.shape[0]
    keys = sel_s.reshape(B, KSUB, LANES)
    idxs = sel_i.reshape(B, KSUB, LANES)
    js, ks = _stage_consts()
    skey, sidx = pl.pallas_call(
        _sort_body,
        in_specs=[
            pl.BlockSpec(memory_space=pltpu.SMEM),
            pl.BlockSpec(memory_space=pltpu.SMEM),
            pl.BlockSpec(),
            pl.BlockSpec(),
        ],
        out_specs=(pl.BlockSpec(), pl.BlockSpec()),
        out_shape=(
            jax.ShapeDtypeStruct((B, KSUB, LANES), jnp.float32),
            jax.ShapeDtypeStruct((B, KSUB, LANES), jnp.int32),
        ),
    )(js, ks, keys, idxs)
    return skey.reshape(B, MAXK), sidx.reshape(B, MAXK)


def _make_sc_gather(n_kp, d_desc, d_kp):
    mesh = plsc.VectorSubcoreMesh(core_axis_name="c", subcore_axis_name="s")

    @functools.partial(
        pl.kernel,
        mesh=mesh,
        compiler_params=pltpu.CompilerParams(needs_layout_passes=False),
        out_type=(
            jax.ShapeDtypeStruct((NW * ROWS_PER_W, d_desc), jnp.float32),
            jax.ShapeDtypeStruct((NW * ROWS_PER_W * d_kp,), jnp.float32),
        ),
        scratch_types=[
            pltpu.VMEM((CPW, CHUNK), jnp.int32),
            pltpu.VMEM((ROWS_PER_W,), jnp.int32),
            pltpu.VMEM((CHUNK, d_desc), jnp.float32),
            pltpu.VMEM((n_kp * d_kp,), jnp.float32),
            pltpu.VMEM((ROWS_PER_W * d_kp,), jnp.float32),
            pltpu.SemaphoreType.DMA,
        ],
    )
    def gather_k(desc_hbm, kp_hbm, gidx_hbm, lidx_hbm, desc_out, kp_out,
                 gidx_v, lidx_v, rows_v, kp_tab_v, kp_res_v, sem_d):
        wid = lax.axis_index("s") * 2 + lax.axis_index("c")
        r = wid // (NW // 8)
        base = wid * ROWS_PER_W
        pltpu.sync_copy(gidx_hbm.at[wid], gidx_v)
        pltpu.sync_copy(lidx_hbm.at[wid], lidx_v)
        pltpu.sync_copy(kp_hbm.at[r], kp_tab_v)
        io16 = lax.iota(jnp.int32, 16)
        for ch in range(CPW):
            cd = pltpu.async_copy(desc_hbm.at[gidx_v.at[ch]], rows_v, sem_d)
            for t in range(CHUNK // 16):
                o = ch * CHUNK + t * 16
                lvec = lidx_v[pl.ds(o, 16)] * 2
                xs = plsc.load_gather(kp_tab_v, [lvec])
                ys = plsc.load_gather(kp_tab_v, [lvec + 1])
                ovec = (io16 + o) * 2
                plsc.store_scatter(kp_res_v, [ovec], xs)
                plsc.store_scatter(kp_res_v, [ovec + 1], ys)
            cd.wait()
            pltpu.sync_copy(rows_v, desc_out.at[pl.ds(base + ch * CHUNK, CHUNK)])
        pltpu.sync_copy(kp_res_v, kp_out.at[pl.ds(base * 2, ROWS_PER_W * 2)])

    return gather_k


def kernel(keypoints, scores, descriptors):
    B, N, D = descriptors.shape

    t, cnt, neq = _threshold(scores)
    compact_k = _make_sc_compact(B, N)
    sel_s, sel_i = compact_k(scores, t, cnt, neq)
    scores_top, idx_top = _topk_sort(sel_s, sel_i)

    gidx = (idx_top + (jnp.arange(B, dtype=jnp.int32) * N)[:, None]).astype(jnp.int32)
    gidx = gidx.reshape(NW, CPW, CHUNK)
    lidx = idx_top.reshape(NW, ROWS_PER_W)

    desc_tab = descriptors.reshape(B * N, D)

    gather_k = _make_sc_gather(N, D, 2)
    desc_flat, kp_flat = gather_k(desc_tab, keypoints.reshape(B, N * 2), gidx, lidx)

    final_kpts = kp_flat.reshape(B, MAXK, 2)
    final_desc = desc_flat.reshape(B, MAXK, D)
    num_valid = jnp.full((B,), MAXK, dtype=jnp.int32)
    return (final_kpts, scores_top, final_desc, num_valid)


# speedup vs baseline: 3.3129x; 1.0167x over previous
import functools

import jax
import jax.numpy as jnp
from jax import lax
from jax.experimental import pallas as pl
from jax.experimental.pallas import tpu as pltpu
from jax.experimental.pallas import tpu_sc as plsc

MAXK = 4096
LANES = 128
LOGK = 12
KSUB = MAXK // LANES

NW = 32
CHUNK = 128
ROWS_PER_W = (8 * MAXK) // NW
CPW = ROWS_PER_W // CHUNK

ONE_F32_BITS = 0x3F800000


def _thresh_body(key_ref, t_ref, cnt_ref, neq_ref):
    kb = lax.bitcast_convert_type(key_ref[...], jnp.int32)
    B = kb.shape[0]
    lo = jnp.full((B, LANES), -1, jnp.int32)
    hi = jnp.full((B, LANES), ONE_F32_BITS, jnp.int32)

    def count_gt(t):
        c = jnp.sum((kb > t[:, None, :]).astype(jnp.int32), axis=1)
        return jnp.sum(c, axis=1, keepdims=True) + jnp.zeros((B, LANES), jnp.int32)

    def step(_, lohi):
        lo, hi = lohi
        mid = (lo + hi) >> 1
        c = count_gt(mid)
        big = c >= MAXK
        return jnp.where(big, mid, lo), jnp.where(big, hi, mid)

    lo, hi = lax.fori_loop(0, 31, step, (lo, hi))
    cnt = count_gt(hi)
    t_ref[...] = hi
    cnt_ref[...] = cnt
    neq_ref[...] = MAXK - cnt


def _threshold(scores):
    B, N = scores.shape
    S = -(-N // LANES)
    padded = jnp.concatenate(
        [scores, jnp.full((B, S * LANES - N), -1.0, scores.dtype)], axis=1
    ).reshape(B, S, LANES)
    return pl.pallas_call(
        _thresh_body,
        out_shape=(
            jax.ShapeDtypeStruct((B, LANES), jnp.int32),
            jax.ShapeDtypeStruct((B, LANES), jnp.int32),
            jax.ShapeDtypeStruct((B, LANES), jnp.int32),
        ),
    )(padded)


def _make_sc_compact(B, N):
    mesh = plsc.VectorSubcoreMesh(core_axis_name="c", subcore_axis_name="s")
    nv = N // 16

    @functools.partial(
        pl.kernel,
        mesh=mesh,
        compiler_params=pltpu.CompilerParams(needs_layout_passes=False),
        out_type=(
            jax.ShapeDtypeStruct((B, MAXK), jnp.float32),
            jax.ShapeDtypeStruct((B, MAXK), jnp.int32),
        ),
        scratch_types=[
            pltpu.VMEM((N,), jnp.float32),
            pltpu.VMEM((LANES,), jnp.int32),
            pltpu.VMEM((LANES,), jnp.int32),
            pltpu.VMEM((LANES,), jnp.int32),
            pltpu.VMEM((MAXK,), jnp.float32),
            pltpu.VMEM((MAXK,), jnp.int32),
        ],
    )
    def compact_k(s_hbm, t_hbm, cnt_hbm, neq_hbm, sel_s_hbm, sel_i_hbm,
                  row_v, t_v, cnt_v, neq_v, outs_v, outi_v):
        wid = lax.axis_index("s") * 2 + lax.axis_index("c")

        @pl.when(wid < B)
        def _():
            pltpu.sync_copy(s_hbm.at[wid], row_v)
            pltpu.sync_copy(t_hbm.at[wid], t_v)
            pltpu.sync_copy(cnt_hbm.at[wid], cnt_v)
            pltpu.sync_copy(neq_hbm.at[wid], neq_v)
            z16 = jnp.zeros((16,), jnp.int32)
            tsp = t_v[pl.ds(0, 16)]
            cntsp = cnt_v[pl.ds(0, 16)]
            neqsp = neq_v[pl.ds(0, 16)]
            io16 = lax.iota(jnp.int32, 16)

            def body(i, carry):
                offg, offe = carry
                sl = row_v[pl.ds(pl.multiple_of(i * 16, 16), 16)]
                kb = plsc.bitcast(sl, jnp.int32)
                ivec = io16 + i * 16
                m = kb > tsp
                inc = plsc.cumsum(jnp.where(m, 1, 0))
                pos = offg + inc - 1
                plsc.store_scatter(outs_v, [pos], sl, mask=m)
                plsc.store_scatter(outi_v, [pos], ivec, mask=m)
                m2 = kb == tsp
                inc2 = plsc.cumsum(jnp.where(m2, 1, 0))
                rank = offe + inc2 - 1
                m2w = m2 & (rank < neqsp)
                pos2 = cntsp + rank
                plsc.store_scatter(outs_v, [pos2], sl, mask=m2w)
                plsc.store_scatter(outi_v, [pos2], ivec, mask=m2w)
                return (offg + plsc.all_reduce_population_count(m),
                        offe + plsc.all_reduce_population_count(m2))

            lax.fori_loop(0, nv, body, (z16, z16))
            pltpu.sync_copy(outs_v, sel_s_hbm.at[wid])
            pltpu.sync_copy(outi_v, sel_i_hbm.at[wid])

    return compact_k


def _stage_consts():
    js, ks = [], []
    for p in range(1, LOGK + 1):
        k = 1 << p
        for q in range(p - 1, -1, -1):
            js.append(1 << q)
            ks.append(k)
    return jnp.array(js, jnp.int32), jnp.array(ks, jnp.int32)


def _partner_axis(x, sh, n, axis, is_left):
    up = pltpu.roll(x, (n - sh) & (n - 1), axis)
    dn = pltpu.roll(x, sh, axis)
    return jnp.where(is_left, up, dn)


def _sort_body(j_ref, k_ref, key_ref, idx_ref, out_key_ref, out_idx_ref):
    shape = key_ref.shape
    s_io = lax.broadcasted_iota(jnp.int32, shape, 1)
    l_io = lax.broadcasted_iota(jnp.int32, shape, 2)
    i_io = s_io * LANES + l_io
    out_key_ref[...] = key_ref[...]
    out_idx_ref[...] = idx_ref[...]

    def exchange(key, idx, pk, pi, is_left, d):
        prec = (key > pk) | ((key == pk) & (idx < pi))
        keep = prec == (is_left == d)
        return jnp.where(keep, key, pk), jnp.where(keep, idx, pi)

    def lane_stage(key, idx, j, is_left, d):
        pk = _partner_axis(key, j, LANES, 2, is_left)
        pi = _partner_axis(idx, j, LANES, 2, is_left)
        return exchange(key, idx, pk, pi, is_left, d)

    def sub_stage(key, idx, j, is_left, d):
        jr = j >> 7
        pk = _partner_axis(key, jr, KSUB, 1, is_left)
        pi = _partner_axis(idx, jr, KSUB, 1, is_left)
        return exchange(key, idx, pk, pi, is_left, d)

    def stage(s, _):
        j = j_ref[s]
        k = k_ref[s]
        is_left = (i_io & j) == 0
        d = (i_io & k) == 0
        key = out_key_ref[...]
        idx = out_idx_ref[...]
        nk, ni = lax.cond(j < LANES, lane_stage, sub_stage,
                          key, idx, j, is_left, d)
        out_key_ref[...] = nk
        out_idx_ref[...] = ni
        return _

    lax.fori_loop(0, j_ref.shape[0], stage, None)


def _topk_sort(sel_s, sel_i):
    B = sel_s.shape[0]
    keys = sel_s.reshape(B, KSUB, LANES)
    idxs = sel_i.reshape(B, KSUB, LANES)
    js, ks = _stage_consts()
    skey, sidx = pl.pallas_call(
        _sort_body,
        in_specs=[
            pl.BlockSpec(memory_space=pltpu.SMEM),
            pl.BlockSpec(memory_space=pltpu.SMEM),
            pl.BlockSpec(),
            pl.BlockSpec(),
        ],
        out_specs=(pl.BlockSpec(), pl.BlockSpec()),
        out_shape=(
            jax.ShapeDtypeStruct((B, KSUB, LANES), jnp.float32),
            jax.ShapeDtypeStruct((B, KSUB, LANES), jnp.int32),
        ),
    )(js, ks, keys, idxs)
    return skey.reshape(B, MAXK), sidx.reshape(B, MAXK)


def _make_sc_gather(n_kp, d_desc, d_kp):
    mesh = plsc.VectorSubcoreMesh(core_axis_name="c", subcore_axis_name="s")

    @functools.partial(
        pl.kernel,
        mesh=mesh,
        compiler_params=pltpu.CompilerParams(needs_layout_passes=False),
        out_type=(
            jax.ShapeDtypeStruct((NW * ROWS_PER_W, d_desc), jnp.float32),
            jax.ShapeDtypeStruct((NW * ROWS_PER_W * d_kp,), jnp.float32),
        ),
        scratch_types=[
            pltpu.VMEM((CPW, CHUNK), jnp.int32),
            pltpu.VMEM((ROWS_PER_W,), jnp.int32),
            pltpu.VMEM((CHUNK, d_desc), jnp.float32),
            pltpu.VMEM((CHUNK, d_desc), jnp.float32),
            pltpu.VMEM((n_kp * d_kp,), jnp.float32),
            pltpu.VMEM((ROWS_PER_W * d_kp,), jnp.float32),
            pltpu.SemaphoreType.DMA,
            pltpu.SemaphoreType.DMA,
        ],
    )
    def gather_k(desc_hbm, kp_hbm, gidx_hbm, lidx_hbm, desc_out, kp_out,
                 gidx_v, lidx_v, rows_v0, rows_v1, kp_tab_v, kp_res_v,
                 sem0, sem1):
        wid = lax.axis_index("s") * 2 + lax.axis_index("c")
        r = wid // (NW // 8)
        base = wid * ROWS_PER_W
        pltpu.sync_copy(gidx_hbm.at[wid], gidx_v)
        pltpu.sync_copy(lidx_hbm.at[wid], lidx_v)
        pltpu.sync_copy(kp_hbm.at[r], kp_tab_v)
        io16 = lax.iota(jnp.int32, 16)
        rows = (rows_v0, rows_v1)
        sems = (sem0, sem1)
        cds = [None, None]
        cds[0] = pltpu.async_copy(desc_hbm.at[gidx_v.at[0]], rows[0], sems[0])
        for ch in range(CPW):
            nxt = ch + 1
            if nxt < CPW:
                cds[nxt % 2] = pltpu.async_copy(
                    desc_hbm.at[gidx_v.at[nxt]], rows[nxt % 2], sems[nxt % 2])
            for t in range(CHUNK // 16):
                o = ch * CHUNK + t * 16
                lvec = lidx_v[pl.ds(o, 16)] * 2
                xs = plsc.load_gather(kp_tab_v, [lvec])
                ys = plsc.load_gather(kp_tab_v, [lvec + 1])
                ovec = (io16 + o) * 2
                plsc.store_scatter(kp_res_v, [ovec], xs)
                plsc.store_scatter(kp_res_v, [ovec + 1], ys)
            cds[ch % 2].wait()
            pltpu.sync_copy(rows[ch % 2],
                            desc_out.at[pl.ds(base + ch * CHUNK, CHUNK)])
        pltpu.sync_copy(kp_res_v, kp_out.at[pl.ds(base * 2, ROWS_PER_W * 2)])

    return gather_k


def kernel(keypoints, scores, descriptors):
    B, N, D = descriptors.shape

    t, cnt, neq = _threshold(scores)
    compact_k = _make_sc_compact(B, N)
    sel_s, sel_i = compact_k(scores, t, cnt, neq)
    scores_top, idx_top = _topk_sort(sel_s, sel_i)

    gidx = (idx_top + (jnp.arange(B, dtype=jnp.int32) * N)[:, None]).astype(jnp.int32)
    gidx = gidx.reshape(NW, CPW, CHUNK)
    lidx = idx_top.reshape(NW, ROWS_PER_W)

    desc_tab = descriptors.reshape(B * N, D)

    gather_k = _make_sc_gather(N, D, 2)
    desc_flat, kp_flat = gather_k(desc_tab, keypoints.reshape(B, N * 2), gidx, lidx)

    final_kpts = kp_flat.reshape(B, MAXK, 2)
    final_desc = desc_flat.reshape(B, MAXK, D)
    num_valid = jnp.full((B,), MAXK, dtype=jnp.int32)
    return (final_kpts, scores_top, final_desc, num_valid)
